# baseline (speedup 1.0000x reference)
import functools

import jax
import jax.numpy as jnp
from jax import lax
from jax.experimental import pallas as pl
from jax.experimental.pallas import tpu as pltpu
from jax.experimental.pallas import tpu_sc as plsc

N = 4096
K = 32
CIN = 128
COUT = 128
TILE = 512
NW = 32


def _dist_kernel(p1_ref, p2_ref, out_ref):
    p1 = p1_ref[0]
    p2 = p2_ref[0]
    dot = lax.dot_general(p1, p2, (((1,), (1,)), ((), ())),
                          preferred_element_type=jnp.float32)
    n1 = jnp.sum(p1 * p1, axis=1)[:, None]
    n2 = jnp.sum(p2 * p2, axis=1)[None, :]
    out_ref[0] = n1 + n2 - 2.0 * dot


def _dist_matrix(P1, P2):
    B = P1.shape[0]
    return pl.pallas_call(
        _dist_kernel,
        grid=(B, N // TILE),
        in_specs=[
            pl.BlockSpec((1, TILE, 3), lambda b, i: (b, i, 0)),
            pl.BlockSpec((1, N, 3), lambda b, i: (b, 0, 0)),
        ],
        out_specs=pl.BlockSpec((1, TILE, N), lambda b, i: (b, i, 0)),
        out_shape=jax.ShapeDtypeStruct((B, N, N), jnp.float32),
    )(P1, P2)


TARGET = 56.0


def _gh_kernel(p1_ref, p2_ref, x1_ref, s2_ref, w_ref, b_ref, g_ref, h_ref,
               t0_ref):
    w = w_ref[...]
    ws = w[:, :COUT]
    wx = w[:, COUT:COUT + CIN]
    wd = w[:, COUT + CIN:]
    s2 = s2_ref[0]
    x1 = x1_ref[0]
    p1 = p1_ref[0]
    p2 = p2_ref[0]
    g = lax.dot_general(s2, ws, (((0,), (1,)), ((), ())),
                        preferred_element_type=jnp.float32)
    g = g + lax.dot_general(p2, wd, (((1,), (1,)), ((), ())),
                            preferred_element_type=jnp.float32)
    g_ref[0] = g
    h = lax.dot_general(wx, x1, (((1,), (0,)), ((), ())),
                        preferred_element_type=jnp.float32)
    h = h - lax.dot_general(wd, p1, (((1,), (1,)), ((), ())),
                            preferred_element_type=jnp.float32)
    h_ref[0] = h + b_ref[...][:, None]
    r = jnp.full((N,), 0.148, jnp.float32)
    t = r * r
    for _ in range(2):
        f = jnp.ones((N,), jnp.float32)
        for a in range(3):
            xa = p1[:, a]
            f = f * ((jnp.minimum(xa + r, 1.0) - jnp.maximum(xa - r, 0.0))
                     / (2.0 * r))
        arg = TARGET / (4096.0 * 4.18879 * jnp.maximum(f, 0.125))
        t = jnp.exp(jnp.log(arg) * (2.0 / 3.0))
        r = jnp.sqrt(t)
    t0_ref[0, 0] = jnp.clip(t, 0.005, 0.25)


def _gh(P1, P2, X1, S2, W, b):
    B = P1.shape[0]
    return pl.pallas_call(
        _gh_kernel,
        grid=(B,),
        in_specs=[
            pl.BlockSpec((1, N, 3), lambda bb: (bb, 0, 0)),
            pl.BlockSpec((1, N, 3), lambda bb: (bb, 0, 0)),
            pl.BlockSpec((1, CIN, N), lambda bb: (bb, 0, 0)),
            pl.BlockSpec((1, COUT, N), lambda bb: (bb, 0, 0)),
            pl.BlockSpec(W.shape, lambda bb: (0, 0)),
            pl.BlockSpec(b.shape, lambda bb: (0,)),
        ],
        out_specs=[
            pl.BlockSpec((1, N, COUT), lambda bb: (bb, 0, 0)),
            pl.BlockSpec((1, COUT, N), lambda bb: (bb, 0, 0)),
            pl.BlockSpec((1, 1, N), lambda bb: (bb, 0, 0)),
        ],
        out_shape=[
            jax.ShapeDtypeStruct((B, N, COUT), jnp.float32),
            jax.ShapeDtypeStruct((B, COUT, N), jnp.float32),
            jax.ShapeDtypeStruct((B, 1, N), jnp.float32),
        ],
    )(P1, P2, X1, S2, W, b)


def _final_kernel(m_ref, h_ref, out_ref):
    m = m_ref[0]
    out_ref[0] = jnp.maximum(m.T + h_ref[0], 0.0)


def _final(M, H):
    B = M.shape[0]
    return pl.pallas_call(
        _final_kernel,
        grid=(B, N // TILE),
        in_specs=[
            pl.BlockSpec((1, TILE, COUT), lambda b, i: (b, i, 0)),
            pl.BlockSpec((1, COUT, TILE), lambda b, i: (b, 0, i)),
        ],
        out_specs=pl.BlockSpec((1, COUT, TILE), lambda b, i: (b, 0, i)),
        out_shape=jax.ShapeDtypeStruct((B, COUT, N), jnp.float32),
    )(M, H)


def _merge16(ak, av, bk, bv):
    rk = jnp.flip(bk, 0)
    rv = jnp.flip(bv, 0)
    cm = ak <= rk
    lk = jnp.where(cm, ak, rk)
    lv = jnp.where(cm, av, rv)
    hk = jnp.where(cm, rk, ak)
    hv = jnp.where(cm, rv, av)
    return plsc.sort_key_val(lk, lv) + plsc.sort_key_val(hk, hv)


def _merge32_low(a, b):
    ak0, av0, ak1, av1 = a
    bk0, bv0, bk1, bv1 = b
    r0k = jnp.flip(bk1, 0)
    r0v = jnp.flip(bv1, 0)
    r1k = jnp.flip(bk0, 0)
    r1v = jnp.flip(bv0, 0)
    c0 = ak0 <= r0k
    l0k = jnp.where(c0, ak0, r0k)
    l0v = jnp.where(c0, av0, r0v)
    c1 = ak1 <= r1k
    l1k = jnp.where(c1, ak1, r1k)
    l1v = jnp.where(c1, av1, r1v)
    cp = l0k <= l1k
    pk = jnp.where(cp, l0k, l1k)
    pv = jnp.where(cp, l0v, l1v)
    qk = jnp.where(cp, l1k, l0k)
    qv = jnp.where(cp, l1v, l0v)
    return plsc.sort_key_val(pk, pv) + plsc.sort_key_val(qk, qv)


def _sc_topk_gathermax(Dflat, Gflat, t0row):
    BN = Dflat.shape[0]
    qpw = BN // NW
    mesh = plsc.VectorSubcoreMesh(core_axis_name="c", subcore_axis_name="s")

    @functools.partial(
        pl.kernel,
        out_type=jax.ShapeDtypeStruct((BN, COUT), jnp.float32),
        mesh=mesh,
        compiler_params=pltpu.CompilerParams(needs_layout_passes=False),
        scratch_types=[
            pltpu.VMEM((N,), jnp.float32),
            pltpu.VMEM((N,), jnp.float32),
            pltpu.VMEM((N,), jnp.int32),
            pltpu.VMEM((K,), jnp.int32),
            pltpu.VMEM((K,), jnp.int32),
            pltpu.VMEM((K, COUT), jnp.float32),
            pltpu.VMEM((K, COUT), jnp.float32),
            pltpu.VMEM((COUT,), jnp.float32),
            pltpu.VMEM((COUT,), jnp.float32),
            pltpu.VMEM((256,), jnp.float32),
            pltpu.SemaphoreType.DMA,
            pltpu.SemaphoreType.DMA,
            pltpu.SemaphoreType.DMA,
            pltpu.SemaphoreType.DMA,
            pltpu.SemaphoreType.DMA,
        ],
    )
    def sck(d_hbm, g_hbm, t0_hbm, out_hbm, d0, d1, clist, i0, i1, g0, g1,
            m0, m1, t0buf, sem_d0, sem_d1, sem_g, sem_o0, sem_o1):
        wid = lax.axis_index("s") * 2 + lax.axis_index("c")
        base = wid * qpw
        lanes = lax.iota(jnp.int32, 16)
        pltpu.sync_copy(t0_hbm.at[pl.ds(base, qpw)], t0buf)
        dbufs = (d0, d1)
        idxs = (i0, i1)
        gbufs = (g0, g1)
        mbufs = (m0, m1)
        dsems = (sem_d0, sem_d1)
        osems = (sem_o0, sem_o1)

        lanebase = lanes * (N // 16)

        def select(dref, iref, q, step):
            boff = (q // N) * N

            def collect(t0):
                def cbody(c, off):
                    d = dref[pl.ds(c * 16, 16)]
                    m = d <= t0
                    plsc.store_scatter(clist, [off], c * 16 + lanes, mask=m)
                    return off + m.astype(jnp.int32)
                off = lax.fori_loop(0, N // 16, cbody, lanebase, unroll=8)
                return off - lanebase

            t0i = plsc.load_gather(t0buf, [jnp.full((16,), step, jnp.int32)])
            cnt0 = collect(t0i)

            def w_cond(st):
                return jnp.sum(st[1]) < K

            def w_body(st):
                t0 = st[0] * 4.0
                return (t0, collect(t0))

            _, cntv = lax.while_loop(w_cond, w_body, (t0i, cnt0))

            inf = jnp.full((16,), jnp.inf, jnp.float32)
            zero = jnp.zeros((16,), jnp.int32)
            nch = jnp.max(cntv)

            def fbody(c, st):
                k0, v0, k1, v1 = st
                valid = c < cntv
                cidx = plsc.load_gather(clist, [lanebase + c], mask=valid)
                keys = plsc.load_gather(dref, [cidx], mask=valid)
                keys = jnp.where(valid, keys, jnp.inf)
                sk, sv = plsc.sort_key_val(keys, cidx)
                rk = jnp.flip(sk, 0)
                rv = jnp.flip(sv, 0)
                cm = k0 <= rk
                a0k, a0v = plsc.sort_key_val(jnp.where(cm, k0, rk),
                                             jnp.where(cm, v0, rv))
                a1k, a1v = plsc.sort_key_val(jnp.where(cm, rk, k0),
                                             jnp.where(cm, rv, v0))
                rk1 = jnp.flip(k1, 0)
                rv1 = jnp.flip(v1, 0)
                cm2 = a1k <= rk1
                b0k, b0v = plsc.sort_key_val(jnp.where(cm2, a1k, rk1),
                                             jnp.where(cm2, a1v, rv1))
                return (a0k, a0v, b0k, b0v)

            _, v0, _, v1 = lax.fori_loop(0, nch, fbody, (inf, zero, inf, zero))
            iref[pl.ds(0, 16)] = v0 + boff
            iref[pl.ds(16, 16)] = v1 + boff

        def maxred(gref, mref):
            accs = [jnp.full((16,), -jnp.inf, jnp.float32)
                    for _ in range(COUT // 16)]
            for r in range(K):
                for g in range(COUT // 16):
                    accs[g] = jnp.maximum(accs[g], gref[r, pl.ds(g * 16, 16)])
            for g in range(COUT // 16):
                mref[pl.ds(g * 16, 16)] = accs[g]

        pltpu.async_copy(d_hbm.at[base], d0, sem_d0)

        def outer(i2, carry):
            for s in (0, 1):
                step = i2 * 2 + s
                q = base + step

                @pl.when(step < qpw)
                def _():
                    pltpu.make_async_copy(d_hbm.at[q], dbufs[s],
                                          dsems[s]).wait()

                    @pl.when(step + 1 < qpw)
                    def _():
                        pltpu.async_copy(d_hbm.at[q + 1], dbufs[1 - s],
                                         dsems[1 - s])

                    select(dbufs[s], idxs[s], q, step)
                    pltpu.async_copy(g_hbm.at[idxs[s]], gbufs[s], sem_g)

                @pl.when((step >= 1) & (step <= qpw))
                def _():
                    pltpu.make_async_copy(g_hbm.at[idxs[1 - s]],
                                          gbufs[1 - s], sem_g).wait()

                    @pl.when(step >= 3)
                    def _():
                        pltpu.make_async_copy(mbufs[1 - s], out_hbm.at[base],
                                              osems[1 - s]).wait()

                    maxred(gbufs[1 - s], mbufs[1 - s])
                    pltpu.async_copy(mbufs[1 - s], out_hbm.at[q - 1],
                                     osems[1 - s])
            return carry

        lax.fori_loop(0, qpw // 2 + 1, outer, 0)
        pltpu.make_async_copy(m0, out_hbm.at[base], sem_o0).wait()
        pltpu.make_async_copy(m1, out_hbm.at[base], sem_o1).wait()

    return sck(Dflat, Gflat, t0row)


def kernel(P1, P2, X1, S2, W, b):
    B = P1.shape[0]
    D = _dist_matrix(P1, P2)
    G, H, T0q = _gh(P1, P2, X1, S2, W, b)
    M = _sc_topk_gathermax(D.reshape(B * N, N), G.reshape(B * N, COUT),
                           T0q.reshape(B * N))
    return _final(M.reshape(B, N, COUT), H)


# speedup vs baseline: 1.2911x; 1.2911x over previous
import functools

import jax
import jax.numpy as jnp
from jax import lax
from jax.experimental import pallas as pl
from jax.experimental.pallas import tpu as pltpu
from jax.experimental.pallas import tpu_sc as plsc

N = 4096
K = 32
CIN = 128
COUT = 128
TILE = 512
NW = 32


NCH = N // 16


def _dist_kernel(p1_ref, p2_ref, out_ref, cmin_ref):
    p1 = p1_ref[0]
    p2 = p2_ref[0]
    dot = lax.dot_general(p1, p2, (((1,), (1,)), ((), ())),
                          preferred_element_type=jnp.float32)
    n1 = jnp.sum(p1 * p1, axis=1)[:, None]
    n2 = jnp.sum(p2 * p2, axis=1)[None, :]
    d = n1 + n2 - 2.0 * dot
    out_ref[0] = d
    cm = jnp.minimum(d[:, 0:NCH], d[:, NCH:2 * NCH])
    for e in range(2, 16):
        cm = jnp.minimum(cm, d[:, e * NCH:(e + 1) * NCH])
    cmin_ref[0] = cm


def _dist_matrix(P1, P2):
    B = P1.shape[0]
    return pl.pallas_call(
        _dist_kernel,
        grid=(B, N // TILE),
        in_specs=[
            pl.BlockSpec((1, TILE, 3), lambda b, i: (b, i, 0)),
            pl.BlockSpec((1, N, 3), lambda b, i: (b, 0, 0)),
        ],
        out_specs=[
            pl.BlockSpec((1, TILE, N), lambda b, i: (b, i, 0)),
            pl.BlockSpec((1, TILE, NCH), lambda b, i: (b, i, 0)),
        ],
        out_shape=[
            jax.ShapeDtypeStruct((B, N, N), jnp.float32),
            jax.ShapeDtypeStruct((B, N, NCH), jnp.float32),
        ],
    )(P1, P2)


TARGET = 56.0


def _gh_kernel(p1_ref, p2_ref, x1_ref, s2_ref, w_ref, b_ref, g_ref, h_ref,
               t0_ref):
    w = w_ref[...]
    ws = w[:, :COUT]
    wx = w[:, COUT:COUT + CIN]
    wd = w[:, COUT + CIN:]
    s2 = s2_ref[0]
    x1 = x1_ref[0]
    p1 = p1_ref[0]
    p2 = p2_ref[0]
    g = lax.dot_general(s2, ws, (((0,), (1,)), ((), ())),
                        preferred_element_type=jnp.float32)
    g = g + lax.dot_general(p2, wd, (((1,), (1,)), ((), ())),
                            preferred_element_type=jnp.float32)
    g_ref[0] = g
    h = lax.dot_general(wx, x1, (((1,), (0,)), ((), ())),
                        preferred_element_type=jnp.float32)
    h = h - lax.dot_general(wd, p1, (((1,), (1,)), ((), ())),
                            preferred_element_type=jnp.float32)
    h_ref[0] = h + b_ref[...][:, None]
    r = jnp.full((N,), 0.148, jnp.float32)
    t = r * r
    for _ in range(2):
        f = jnp.ones((N,), jnp.float32)
        for a in range(3):
            xa = p1[:, a]
            f = f * ((jnp.minimum(xa + r, 1.0) - jnp.maximum(xa - r, 0.0))
                     / (2.0 * r))
        arg = TARGET / (4096.0 * 4.18879 * jnp.maximum(f, 0.125))
        t = jnp.exp(jnp.log(arg) * (2.0 / 3.0))
        r = jnp.sqrt(t)
    t0_ref[0, 0] = jnp.clip(t, 0.005, 0.25)


def _gh(P1, P2, X1, S2, W, b):
    B = P1.shape[0]
    return pl.pallas_call(
        _gh_kernel,
        grid=(B,),
        in_specs=[
            pl.BlockSpec((1, N, 3), lambda bb: (bb, 0, 0)),
            pl.BlockSpec((1, N, 3), lambda bb: (bb, 0, 0)),
            pl.BlockSpec((1, CIN, N), lambda bb: (bb, 0, 0)),
            pl.BlockSpec((1, COUT, N), lambda bb: (bb, 0, 0)),
            pl.BlockSpec(W.shape, lambda bb: (0, 0)),
            pl.BlockSpec(b.shape, lambda bb: (0,)),
        ],
        out_specs=[
            pl.BlockSpec((1, N, COUT), lambda bb: (bb, 0, 0)),
            pl.BlockSpec((1, COUT, N), lambda bb: (bb, 0, 0)),
            pl.BlockSpec((1, 1, N), lambda bb: (bb, 0, 0)),
        ],
        out_shape=[
            jax.ShapeDtypeStruct((B, N, COUT), jnp.float32),
            jax.ShapeDtypeStruct((B, COUT, N), jnp.float32),
            jax.ShapeDtypeStruct((B, 1, N), jnp.float32),
        ],
    )(P1, P2, X1, S2, W, b)


def _final_kernel(m_ref, h_ref, out_ref):
    m = m_ref[0]
    out_ref[0] = jnp.maximum(m.T + h_ref[0], 0.0)


def _final(M, H):
    B = M.shape[0]
    return pl.pallas_call(
        _final_kernel,
        grid=(B, N // TILE),
        in_specs=[
            pl.BlockSpec((1, TILE, COUT), lambda b, i: (b, i, 0)),
            pl.BlockSpec((1, COUT, TILE), lambda b, i: (b, 0, i)),
        ],
        out_specs=pl.BlockSpec((1, COUT, TILE), lambda b, i: (b, 0, i)),
        out_shape=jax.ShapeDtypeStruct((B, COUT, N), jnp.float32),
    )(M, H)


def _merge16(ak, av, bk, bv):
    rk = jnp.flip(bk, 0)
    rv = jnp.flip(bv, 0)
    cm = ak <= rk
    lk = jnp.where(cm, ak, rk)
    lv = jnp.where(cm, av, rv)
    hk = jnp.where(cm, rk, ak)
    hv = jnp.where(cm, rv, av)
    return plsc.sort_key_val(lk, lv) + plsc.sort_key_val(hk, hv)


def _merge32_low(a, b):
    ak0, av0, ak1, av1 = a
    bk0, bv0, bk1, bv1 = b
    r0k = jnp.flip(bk1, 0)
    r0v = jnp.flip(bv1, 0)
    r1k = jnp.flip(bk0, 0)
    r1v = jnp.flip(bv0, 0)
    c0 = ak0 <= r0k
    l0k = jnp.where(c0, ak0, r0k)
    l0v = jnp.where(c0, av0, r0v)
    c1 = ak1 <= r1k
    l1k = jnp.where(c1, ak1, r1k)
    l1v = jnp.where(c1, av1, r1v)
    cp = l0k <= l1k
    pk = jnp.where(cp, l0k, l1k)
    pv = jnp.where(cp, l0v, l1v)
    qk = jnp.where(cp, l1k, l0k)
    qv = jnp.where(cp, l1v, l0v)
    return plsc.sort_key_val(pk, pv) + plsc.sort_key_val(qk, qv)


def _sc_topk_gathermax(Dflat, Gflat, Cflat, t0row):
    BN = Dflat.shape[0]
    qpw = BN // NW
    mesh = plsc.VectorSubcoreMesh(core_axis_name="c", subcore_axis_name="s")

    @functools.partial(
        pl.kernel,
        out_type=jax.ShapeDtypeStruct((BN, COUT), jnp.float32),
        mesh=mesh,
        compiler_params=pltpu.CompilerParams(needs_layout_passes=False),
        scratch_types=[
            pltpu.VMEM((N,), jnp.float32),
            pltpu.VMEM((N,), jnp.float32),
            pltpu.VMEM((NCH,), jnp.float32),
            pltpu.VMEM((NCH,), jnp.float32),
            pltpu.VMEM((NCH,), jnp.int32),
            pltpu.VMEM((N,), jnp.int32),
            pltpu.VMEM((K,), jnp.int32),
            pltpu.VMEM((K,), jnp.int32),
            pltpu.VMEM((K, COUT), jnp.float32),
            pltpu.VMEM((K, COUT), jnp.float32),
            pltpu.VMEM((COUT,), jnp.float32),
            pltpu.VMEM((COUT,), jnp.float32),
            pltpu.VMEM((256,), jnp.float32),
            pltpu.SemaphoreType.DMA,
            pltpu.SemaphoreType.DMA,
            pltpu.SemaphoreType.DMA,
            pltpu.SemaphoreType.DMA,
            pltpu.SemaphoreType.DMA,
            pltpu.SemaphoreType.DMA,
            pltpu.SemaphoreType.DMA,
        ],
    )
    def sck(d_hbm, g_hbm, c_hbm, t0_hbm, out_hbm, d0, d1, c0, c1, chlist,
            clist, i0, i1, g0, g1, m0, m1, t0buf, sem_d0, sem_d1, sem_c0,
            sem_c1, sem_g, sem_o0, sem_o1):
        wid = lax.axis_index("s") * 2 + lax.axis_index("c")
        base = wid * qpw
        lanes = lax.iota(jnp.int32, 16)
        pltpu.sync_copy(t0_hbm.at[pl.ds(base, qpw)], t0buf)
        dbufs = (d0, d1)
        cbufs = (c0, c1)
        idxs = (i0, i1)
        gbufs = (g0, g1)
        mbufs = (m0, m1)
        dsems = (sem_d0, sem_d1)
        csems = (sem_c0, sem_c1)
        osems = (sem_o0, sem_o1)

        lanebase = lanes * (N // 16)
        chbase = lanes * (NCH // 16)

        def select(dref, cref, iref, q, step):
            boff = (q // N) * N

            def collect(t0):
                def l1body(ci, off):
                    cm = cref[pl.ds(ci * 16, 16)]
                    m = cm <= t0
                    plsc.store_scatter(chlist, [off], ci * 16 + lanes, mask=m)
                    return off + m.astype(jnp.int32)
                choff = lax.fori_loop(0, NCH // 16, l1body, chbase, unroll=8)
                chcnt = choff - chbase

                def l2body(p, off):
                    lv = p < chcnt
                    cid = plsc.load_gather(chlist, [chbase + p], mask=lv)
                    o = off
                    for e in range(16):
                        idx = cid + NCH * e
                        d = plsc.load_gather(dref, [idx], mask=lv)
                        m = lv & (d <= t0)
                        plsc.store_scatter(clist, [o], idx, mask=m)
                        o = o + m.astype(jnp.int32)
                    return o

                off = lax.fori_loop(0, jnp.max(chcnt), l2body, lanebase)
                return off - lanebase

            t0i = plsc.load_gather(t0buf, [jnp.full((16,), step, jnp.int32)])
            cnt0 = collect(t0i)

            def w_cond(st):
                return jnp.sum(st[1]) < K

            def w_body(st):
                t0 = st[0] * 4.0
                return (t0, collect(t0))

            _, cntv = lax.while_loop(w_cond, w_body, (t0i, cnt0))

            inf = jnp.full((16,), jnp.inf, jnp.float32)
            zero = jnp.zeros((16,), jnp.int32)
            nch = jnp.max(cntv)

            def fbody(c, st):
                k0, v0, k1, v1 = st
                valid = c < cntv
                cidx = plsc.load_gather(clist, [lanebase + c], mask=valid)
                keys = plsc.load_gather(dref, [cidx], mask=valid)
                keys = jnp.where(valid, keys, jnp.inf)
                sk, sv = plsc.sort_key_val(keys, cidx)
                rk = jnp.flip(sk, 0)
                rv = jnp.flip(sv, 0)
                cm = k0 <= rk
                a0k, a0v = plsc.sort_key_val(jnp.where(cm, k0, rk),
                                             jnp.where(cm, v0, rv))
                a1k, a1v = plsc.sort_key_val(jnp.where(cm, rk, k0),
                                             jnp.where(cm, rv, v0))
                rk1 = jnp.flip(k1, 0)
                rv1 = jnp.flip(v1, 0)
                cm2 = a1k <= rk1
                b0k, b0v = plsc.sort_key_val(jnp.where(cm2, a1k, rk1),
                                             jnp.where(cm2, a1v, rv1))
                return (a0k, a0v, b0k, b0v)

            _, v0, _, v1 = lax.fori_loop(0, nch, fbody, (inf, zero, inf, zero))
            iref[pl.ds(0, 16)] = v0 + boff
            iref[pl.ds(16, 16)] = v1 + boff

        def maxred(gref, mref):
            accs = [jnp.full((16,), -jnp.inf, jnp.float32)
                    for _ in range(COUT // 16)]
            for r in range(K):
                for g in range(COUT // 16):
                    accs[g] = jnp.maximum(accs[g], gref[r, pl.ds(g * 16, 16)])
            for g in range(COUT // 16):
                mref[pl.ds(g * 16, 16)] = accs[g]

        pltpu.async_copy(d_hbm.at[base], d0, sem_d0)
        pltpu.async_copy(c_hbm.at[base], c0, sem_c0)

        def outer(i2, carry):
            for s in (0, 1):
                step = i2 * 2 + s
                q = base + step

                @pl.when(step < qpw)
                def _():
                    pltpu.make_async_copy(d_hbm.at[q], dbufs[s],
                                          dsems[s]).wait()
                    pltpu.make_async_copy(c_hbm.at[q], cbufs[s],
                                          csems[s]).wait()

                    @pl.when(step + 1 < qpw)
                    def _():
                        pltpu.async_copy(d_hbm.at[q + 1], dbufs[1 - s],
                                         dsems[1 - s])
                        pltpu.async_copy(c_hbm.at[q + 1], cbufs[1 - s],
                                         csems[1 - s])

                    select(dbufs[s], cbufs[s], idxs[s], q, step)
                    pltpu.async_copy(g_hbm.at[idxs[s]], gbufs[s], sem_g)

                @pl.when((step >= 1) & (step <= qpw))
                def _():
                    pltpu.make_async_copy(g_hbm.at[idxs[1 - s]],
                                          gbufs[1 - s], sem_g).wait()

                    @pl.when(step >= 3)
                    def _():
                        pltpu.make_async_copy(mbufs[1 - s], out_hbm.at[base],
                                              osems[1 - s]).wait()

                    maxred(gbufs[1 - s], mbufs[1 - s])
                    pltpu.async_copy(mbufs[1 - s], out_hbm.at[q - 1],
                                     osems[1 - s])
            return carry

        lax.fori_loop(0, qpw // 2 + 1, outer, 0)
        pltpu.make_async_copy(m0, out_hbm.at[base], sem_o0).wait()
        pltpu.make_async_copy(m1, out_hbm.at[base], sem_o1).wait()

    return sck(Dflat, Gflat, Cflat, t0row)


def kernel(P1, P2, X1, S2, W, b):
    B = P1.shape[0]
    q3 = jnp.clip(P2 * 8.0, 0.0, 7.0).astype(jnp.int32)

    def _il3(x):
        return (x & 1) | ((x & 2) << 2) | ((x & 4) << 4)

    code = ((_il3(q3[..., 0]) << 2) | (_il3(q3[..., 1]) << 1)
            | _il3(q3[..., 2]))
    sidx = jnp.argsort(code, axis=1).astype(jnp.int32)
    invpos = (jnp.arange(N) % NCH) * 16 + jnp.arange(N) // NCH
    perm = sidx[:, invpos]
    P2p = jnp.take_along_axis(P2, perm[:, :, None], axis=1)
    S2p = jnp.take_along_axis(S2, perm[:, None, :], axis=2)
    D, C = _dist_matrix(P1, P2p)
    G, H, T0q = _gh(P1, P2p, X1, S2p, W, b)
    M = _sc_topk_gathermax(D.reshape(B * N, N), G.reshape(B * N, COUT),
                           C.reshape(B * N, NCH), T0q.reshape(B * N))
    return _final(M.reshape(B, N, COUT), H)


# speedup vs baseline: 1.3142x; 1.0179x over previous
import functools

import jax
import jax.numpy as jnp
from jax import lax
from jax.experimental import pallas as pl
from jax.experimental.pallas import tpu as pltpu
from jax.experimental.pallas import tpu_sc as plsc

N = 4096
K = 32
CIN = 128
COUT = 128
TILE = 512
NW = 32


NCH = N // 16


def _dist_kernel(p1_ref, p2_ref, out_ref, cmin_ref):
    p1 = p1_ref[0]
    p2 = p2_ref[0]
    dot = lax.dot_general(p1, p2, (((1,), (1,)), ((), ())),
                          preferred_element_type=jnp.float32)
    n1 = jnp.sum(p1 * p1, axis=1)[:, None]
    n2 = jnp.sum(p2 * p2, axis=1)[None, :]
    d = n1 + n2 - 2.0 * dot
    out_ref[0] = d
    cm = jnp.minimum(d[:, 0:NCH], d[:, NCH:2 * NCH])
    for e in range(2, 16):
        cm = jnp.minimum(cm, d[:, e * NCH:(e + 1) * NCH])
    cmin_ref[0] = cm


def _dist_matrix(P1, P2):
    B = P1.shape[0]
    return pl.pallas_call(
        _dist_kernel,
        grid=(B, N // TILE),
        in_specs=[
            pl.BlockSpec((1, TILE, 3), lambda b, i: (b, i, 0)),
            pl.BlockSpec((1, N, 3), lambda b, i: (b, 0, 0)),
        ],
        out_specs=[
            pl.BlockSpec((1, TILE, N), lambda b, i: (b, i, 0)),
            pl.BlockSpec((1, TILE, NCH), lambda b, i: (b, i, 0)),
        ],
        out_shape=[
            jax.ShapeDtypeStruct((B, N, N), jnp.float32),
            jax.ShapeDtypeStruct((B, N, NCH), jnp.float32),
        ],
    )(P1, P2)


TARGET = 56.0


def _gh_kernel(p1_ref, p2_ref, x1_ref, s2_ref, w_ref, b_ref, g_ref, h_ref,
               t0_ref):
    w = w_ref[...]
    ws = w[:, :COUT]
    wx = w[:, COUT:COUT + CIN]
    wd = w[:, COUT + CIN:]
    s2 = s2_ref[0]
    x1 = x1_ref[0]
    p1 = p1_ref[0]
    p2 = p2_ref[0]
    g = lax.dot_general(s2, ws, (((0,), (1,)), ((), ())),
                        preferred_element_type=jnp.float32)
    g = g + lax.dot_general(p2, wd, (((1,), (1,)), ((), ())),
                            preferred_element_type=jnp.float32)
    g_ref[0] = g
    h = lax.dot_general(wx, x1, (((1,), (0,)), ((), ())),
                        preferred_element_type=jnp.float32)
    h = h - lax.dot_general(wd, p1, (((1,), (1,)), ((), ())),
                            preferred_element_type=jnp.float32)
    h_ref[0] = h + b_ref[...][:, None]
    r = jnp.full((N,), 0.148, jnp.float32)
    t = r * r
    for _ in range(2):
        f = jnp.ones((N,), jnp.float32)
        for a in range(3):
            xa = p1[:, a]
            f = f * ((jnp.minimum(xa + r, 1.0) - jnp.maximum(xa - r, 0.0))
                     / (2.0 * r))
        arg = TARGET / (4096.0 * 4.18879 * jnp.maximum(f, 0.125))
        t = jnp.exp(jnp.log(arg) * (2.0 / 3.0))
        r = jnp.sqrt(t)
    t0_ref[0, 0] = jnp.clip(t, 0.005, 0.25)


def _gh(P1, P2, X1, S2, W, b):
    B = P1.shape[0]
    return pl.pallas_call(
        _gh_kernel,
        grid=(B,),
        in_specs=[
            pl.BlockSpec((1, N, 3), lambda bb: (bb, 0, 0)),
            pl.BlockSpec((1, N, 3), lambda bb: (bb, 0, 0)),
            pl.BlockSpec((1, CIN, N), lambda bb: (bb, 0, 0)),
            pl.BlockSpec((1, COUT, N), lambda bb: (bb, 0, 0)),
            pl.BlockSpec(W.shape, lambda bb: (0, 0)),
            pl.BlockSpec(b.shape, lambda bb: (0,)),
        ],
        out_specs=[
            pl.BlockSpec((1, N, COUT), lambda bb: (bb, 0, 0)),
            pl.BlockSpec((1, COUT, N), lambda bb: (bb, 0, 0)),
            pl.BlockSpec((1, 1, N), lambda bb: (bb, 0, 0)),
        ],
        out_shape=[
            jax.ShapeDtypeStruct((B, N, COUT), jnp.float32),
            jax.ShapeDtypeStruct((B, COUT, N), jnp.float32),
            jax.ShapeDtypeStruct((B, 1, N), jnp.float32),
        ],
    )(P1, P2, X1, S2, W, b)


def _final_kernel(m_ref, h_ref, out_ref):
    m = m_ref[0]
    out_ref[0] = jnp.maximum(m.T + h_ref[0], 0.0)


def _final(M, H):
    B = M.shape[0]
    return pl.pallas_call(
        _final_kernel,
        grid=(B, N // TILE),
        in_specs=[
            pl.BlockSpec((1, TILE, COUT), lambda b, i: (b, i, 0)),
            pl.BlockSpec((1, COUT, TILE), lambda b, i: (b, 0, i)),
        ],
        out_specs=pl.BlockSpec((1, COUT, TILE), lambda b, i: (b, 0, i)),
        out_shape=jax.ShapeDtypeStruct((B, COUT, N), jnp.float32),
    )(M, H)


def _merge16(ak, av, bk, bv):
    rk = jnp.flip(bk, 0)
    rv = jnp.flip(bv, 0)
    cm = ak <= rk
    lk = jnp.where(cm, ak, rk)
    lv = jnp.where(cm, av, rv)
    hk = jnp.where(cm, rk, ak)
    hv = jnp.where(cm, rv, av)
    return plsc.sort_key_val(lk, lv) + plsc.sort_key_val(hk, hv)


def _merge32_low(a, b):
    ak0, av0, ak1, av1 = a
    bk0, bv0, bk1, bv1 = b
    r0k = jnp.flip(bk1, 0)
    r0v = jnp.flip(bv1, 0)
    r1k = jnp.flip(bk0, 0)
    r1v = jnp.flip(bv0, 0)
    c0 = ak0 <= r0k
    l0k = jnp.where(c0, ak0, r0k)
    l0v = jnp.where(c0, av0, r0v)
    c1 = ak1 <= r1k
    l1k = jnp.where(c1, ak1, r1k)
    l1v = jnp.where(c1, av1, r1v)
    cp = l0k <= l1k
    pk = jnp.where(cp, l0k, l1k)
    pv = jnp.where(cp, l0v, l1v)
    qk = jnp.where(cp, l1k, l0k)
    qv = jnp.where(cp, l1v, l0v)
    return plsc.sort_key_val(pk, pv) + plsc.sort_key_val(qk, qv)


def _sc_topk_gathermax(Dflat, Gflat, Cflat, t0row):
    BN = Dflat.shape[0]
    qpw = BN // NW
    mesh = plsc.VectorSubcoreMesh(core_axis_name="c", subcore_axis_name="s")

    @functools.partial(
        pl.kernel,
        out_type=jax.ShapeDtypeStruct((BN, COUT), jnp.float32),
        mesh=mesh,
        compiler_params=pltpu.CompilerParams(needs_layout_passes=False),
        scratch_types=[
            pltpu.VMEM((N,), jnp.float32),
            pltpu.VMEM((N,), jnp.float32),
            pltpu.VMEM((NCH,), jnp.float32),
            pltpu.VMEM((NCH,), jnp.float32),
            pltpu.VMEM((NCH,), jnp.int32),
            pltpu.VMEM((N,), jnp.int32),
            pltpu.VMEM((K,), jnp.int32),
            pltpu.VMEM((K,), jnp.int32),
            pltpu.VMEM((K, COUT), jnp.float32),
            pltpu.VMEM((K, COUT), jnp.float32),
            pltpu.VMEM((COUT,), jnp.float32),
            pltpu.VMEM((COUT,), jnp.float32),
            pltpu.VMEM((qpw,), jnp.float32),
            pltpu.SemaphoreType.DMA,
            pltpu.SemaphoreType.DMA,
            pltpu.SemaphoreType.DMA,
            pltpu.SemaphoreType.DMA,
            pltpu.SemaphoreType.DMA,
            pltpu.SemaphoreType.DMA,
            pltpu.SemaphoreType.DMA,
        ],
    )
    def sck(d_hbm, g_hbm, c_hbm, t0_hbm, out_hbm, d0, d1, c0, c1, chlist,
            clist, i0, i1, g0, g1, m0, m1, t0buf, sem_d0, sem_d1, sem_c0,
            sem_c1, sem_g, sem_o0, sem_o1):
        wid = lax.axis_index("s") * 2 + lax.axis_index("c")
        base = wid * qpw
        lanes = lax.iota(jnp.int32, 16)
        pltpu.sync_copy(t0_hbm.at[pl.ds(base, qpw)], t0buf)
        dbufs = (d0, d1)
        cbufs = (c0, c1)
        idxs = (i0, i1)
        gbufs = (g0, g1)
        mbufs = (m0, m1)
        dsems = (sem_d0, sem_d1)
        csems = (sem_c0, sem_c1)
        osems = (sem_o0, sem_o1)

        lanebase = lanes * (N // 16)
        chbase = lanes * (NCH // 16)

        def select(dref, cref, iref, q, step):
            boff = (q // N) * N

            def collect(t0):
                def l1body(ci, off):
                    cm = cref[pl.ds(ci * 16, 16)]
                    m = cm <= t0
                    plsc.store_scatter(chlist, [off], ci * 16 + lanes, mask=m)
                    return off + m.astype(jnp.int32)
                choff = lax.fori_loop(0, NCH // 16, l1body, chbase, unroll=8)
                chcnt = choff - chbase

                def l2body(p, off):
                    lv = p < chcnt
                    cid = plsc.load_gather(chlist, [chbase + p], mask=lv)
                    o = off
                    for e in range(16):
                        idx = cid + NCH * e
                        d = plsc.load_gather(dref, [idx], mask=lv)
                        m = lv & (d <= t0)
                        plsc.store_scatter(clist, [o], idx, mask=m)
                        o = o + m.astype(jnp.int32)
                    return o

                off = lax.fori_loop(0, jnp.max(chcnt), l2body, lanebase)
                return off - lanebase

            t0i = plsc.load_gather(t0buf, [jnp.full((16,), step, jnp.int32)])
            cnt0 = collect(t0i)

            def w_cond(st):
                return jnp.sum(st[1]) < K

            def w_body(st):
                t0 = st[0] * 4.0
                return (t0, collect(t0))

            _, cntv = lax.while_loop(w_cond, w_body, (t0i, cnt0))

            inf = jnp.full((16,), jnp.inf, jnp.float32)
            zero = jnp.zeros((16,), jnp.int32)
            nch = jnp.max(cntv)

            def fbody(c, st):
                k0, v0, k1, v1 = st
                valid = c < cntv
                cidx = plsc.load_gather(clist, [lanebase + c], mask=valid)
                keys = plsc.load_gather(dref, [cidx], mask=valid)
                keys = jnp.where(valid, keys, jnp.inf)
                sk, sv = plsc.sort_key_val(keys, cidx)
                rk = jnp.flip(sk, 0)
                rv = jnp.flip(sv, 0)
                cm = k0 <= rk
                a0k, a0v = plsc.sort_key_val(jnp.where(cm, k0, rk),
                                             jnp.where(cm, v0, rv))
                a1k, a1v = plsc.sort_key_val(jnp.where(cm, rk, k0),
                                             jnp.where(cm, rv, v0))
                rk1 = jnp.flip(k1, 0)
                rv1 = jnp.flip(v1, 0)
                cm2 = a1k <= rk1
                b0k, b0v = plsc.sort_key_val(jnp.where(cm2, a1k, rk1),
                                             jnp.where(cm2, a1v, rv1))
                return (a0k, a0v, b0k, b0v)

            _, v0, _, v1 = lax.fori_loop(0, nch, fbody, (inf, zero, inf, zero))
            iref[pl.ds(0, 16)] = v0 + boff
            iref[pl.ds(16, 16)] = v1 + boff

        def maxred(gref, mref):
            accs = [jnp.full((16,), -jnp.inf, jnp.float32)
                    for _ in range(COUT // 16)]
            for r in range(K):
                for g in range(COUT // 16):
                    accs[g] = jnp.maximum(accs[g], gref[r, pl.ds(g * 16, 16)])
            for g in range(COUT // 16):
                mref[pl.ds(g * 16, 16)] = accs[g]

        pltpu.async_copy(d_hbm.at[base], d0, sem_d0)
        pltpu.async_copy(c_hbm.at[base], c0, sem_c0)

        def outer(i2, carry):
            for s in (0, 1):
                step = i2 * 2 + s
                q = base + step

                @pl.when(step < qpw)
                def _():
                    pltpu.make_async_copy(d_hbm.at[q], dbufs[s],
                                          dsems[s]).wait()
                    pltpu.make_async_copy(c_hbm.at[q], cbufs[s],
                                          csems[s]).wait()

                    @pl.when(step + 1 < qpw)
                    def _():
                        pltpu.async_copy(d_hbm.at[q + 1], dbufs[1 - s],
                                         dsems[1 - s])
                        pltpu.async_copy(c_hbm.at[q + 1], cbufs[1 - s],
                                         csems[1 - s])

                    select(dbufs[s], cbufs[s], idxs[s], q, step)
                    pltpu.async_copy(g_hbm.at[idxs[s]], gbufs[s], sem_g)

                @pl.when((step >= 1) & (step <= qpw))
                def _():
                    pltpu.make_async_copy(g_hbm.at[idxs[1 - s]],
                                          gbufs[1 - s], sem_g).wait()

                    @pl.when(step >= 3)
                    def _():
                        pltpu.make_async_copy(mbufs[1 - s], out_hbm.at[base],
                                              osems[1 - s]).wait()

                    maxred(gbufs[1 - s], mbufs[1 - s])
                    pltpu.async_copy(mbufs[1 - s], out_hbm.at[q - 1],
                                     osems[1 - s])
            return carry

        lax.fori_loop(0, qpw // 2 + 1, outer, 0)
        pltpu.make_async_copy(m0, out_hbm.at[base], sem_o0).wait()
        pltpu.make_async_copy(m1, out_hbm.at[base], sem_o1).wait()

    return sck(Dflat, Gflat, Cflat, t0row)


def kernel(P1, P2, X1, S2, W, b):
    B = P1.shape[0]
    q3 = jnp.clip(P2 * 8.0, 0.0, 7.0).astype(jnp.int32)

    def _il3(x):
        return (x & 1) | ((x & 2) << 2) | ((x & 4) << 4)

    code = ((_il3(q3[..., 0]) << 2) | (_il3(q3[..., 1]) << 1)
            | _il3(q3[..., 2]))
    sidx = jnp.argsort(code, axis=1).astype(jnp.int32)
    invpos = (jnp.arange(N) % NCH) * 16 + jnp.arange(N) // NCH
    perm = sidx[:, invpos]
    P2p = jnp.take_along_axis(P2, perm[:, :, None], axis=1)
    S2p = jnp.take_along_axis(S2, perm[:, None, :], axis=2)
    Ms, Hs = [], []
    for bb in range(B):
        sl = slice(bb, bb + 1)
        D, C = _dist_matrix(P1[sl], P2p[sl])
        G, H, T0q = _gh(P1[sl], P2p[sl], X1[sl], S2p[sl], W, b)
        Ms.append(_sc_topk_gathermax(D.reshape(N, N), G.reshape(N, COUT),
                                     C.reshape(N, NCH), T0q.reshape(N)))
        Hs.append(H)
    M = jnp.stack(Ms)
    return _final(M, jnp.concatenate(Hs, axis=0))
